# baseline (speedup 1.0000x reference)
import functools

import jax
import jax.numpy as jnp
from jax import lax
from jax.experimental import pallas as pl
from jax.experimental.pallas import tpu as pltpu
from jax.experimental.pallas import tpu_sc as plsc

B = 16384
F = 26
E = 128
VF = 1000
R = B * F
NW = 32
RW = R // NW
FH = F // 2
FS = 6
C = 64
NCH = RW // C
NBUF = 8

_mesh = plsc.VectorSubcoreMesh(core_axis_name="c", subcore_axis_name="s")


@functools.partial(
    pl.kernel,
    out_type=jax.ShapeDtypeStruct((R, E), jnp.float32),
    mesh=_mesh,
    scratch_types=[
        pltpu.VMEM((RW,), jnp.int32),
        pltpu.VMEM((NBUF, C, E), jnp.float32),
        pltpu.VMEM_SHARED((FS * VF, E), jnp.float32),
    ]
    + [pltpu.SemaphoreType.DMA] * (2 * NBUF + 1),
)
def _embed(x_hbm, w_hbm, out_hbm, idx_v, rows_v, table_sh, *sems):
    gsem = sems[:NBUF]
    osem = sems[NBUF:2 * NBUF]
    tsem = sems[2 * NBUF]
    cid = lax.axis_index("c")
    sid = lax.axis_index("s")
    wid = cid * 16 + sid

    @pl.when(sid < FS)
    def _():
        pltpu.async_copy(
            w_hbm.at[pl.ds((cid * FH + sid) * VF, VF)],
            table_sh.at[pl.ds(sid * VF, VF)],
            tsem,
        )

    pltpu.sync_copy(x_hbm.at[wid], idx_v)

    lanes = lax.iota(jnp.int32, 16)
    cblock = cid * FH * B
    hbase = cid * FH * VF

    def chunk_q0(g):
        return cblock + (g * 16 + sid) * C

    def transform(g):
        for k in range(C // 16):
            q = chunk_q0(g) + k * 16 + lanes
            l = lax.shift_right_logical(q, 14) - cid * FH
            off = l * VF + jnp.where(l < FS, 0, hbase)
            sl = pl.ds(g * C + k * 16, 16)
            idx_v[sl] = idx_v[sl] + off

    @pl.when(sid < FS)
    def _():
        pltpu.make_async_copy(
            w_hbm.at[pl.ds(0, VF)], table_sh.at[pl.ds(0, VF)], tsem
        ).wait()

    plsc.subcore_barrier()

    def start_gather(g, b):
        l0 = lax.shift_right_logical(chunk_q0(g), 14) - cid * FH

        @pl.when(l0 < FS)
        def _():
            pltpu.async_copy(
                table_sh.at[idx_v.at[pl.ds(g * C, C)]], rows_v.at[b], gsem[b]
            )

        @pl.when(l0 >= FS)
        def _():
            pltpu.async_copy(
                w_hbm.at[idx_v.at[pl.ds(g * C, C)]], rows_v.at[b], gsem[b]
            )

    def wait_gather(b):
        pltpu.make_async_copy(
            w_hbm.at[idx_v.at[pl.ds(0, C)]], rows_v.at[b], gsem[b]
        ).wait()

    def start_out(g, b):
        pltpu.async_copy(
            rows_v.at[b], out_hbm.at[pl.ds(chunk_q0(g), C)], osem[b]
        )

    def wait_out(b):
        pltpu.make_async_copy(
            rows_v.at[b], out_hbm.at[pl.ds(0, C)], osem[b]
        ).wait()

    for b in range(NBUF):
        transform(b)
        start_gather(b, b)

    @pl.loop(0, NCH, step=NBUF)
    def _chunks(g0):
        for b in range(NBUF):
            g = g0 + b
            wait_gather(b)
            start_out(g, b)

            @pl.when(g0 + NBUF < NCH)
            def _():
                transform(g + NBUF)
                wait_out(b)
                start_gather(g + NBUF, b)

    for b in range(NBUF):
        wait_out(b)


def kernel(x, weight):
    xq = (
        x.T.reshape(2, NCH, 16, C)
        .transpose(0, 2, 1, 3)
        .reshape(NW, RW)
    )
    out = _embed(xq, weight)
    return out.reshape(F, B, E).transpose(1, 0, 2)


# speedup vs baseline: 1.0843x; 1.0843x over previous
import functools

import jax
import jax.numpy as jnp
from jax import lax
from jax.experimental import pallas as pl
from jax.experimental.pallas import tpu as pltpu
from jax.experimental.pallas import tpu_sc as plsc

B = 16384
F = 26
E = 128
VF = 1000
R = B * F
NW = 32
RW = R // NW
FH = F // 2
FS = 6
C = 64
NCH = RW // C
NBUF = 8

_mesh = plsc.VectorSubcoreMesh(core_axis_name="c", subcore_axis_name="s")


@functools.partial(
    pl.kernel,
    out_type=jax.ShapeDtypeStruct((R, E), jnp.float32),
    mesh=_mesh,
    scratch_types=[
        pltpu.VMEM((RW,), jnp.int32),
        pltpu.VMEM((NBUF, C, E), jnp.float32),
        pltpu.VMEM_SHARED((FS * VF, E), jnp.float32),
    ]
    + [pltpu.SemaphoreType.DMA] * (2 * NBUF + 1),
)
def _embed(x_hbm, w_hbm, out_hbm, idx_v, rows_v, table_sh, *sems):
    gsem = sems[:NBUF]
    osem = sems[NBUF:2 * NBUF]
    tsem = sems[2 * NBUF]
    cid = lax.axis_index("c")
    sid = lax.axis_index("s")
    wid = cid * 16 + sid

    @pl.when(sid < FS)
    def _():
        pltpu.async_copy(
            w_hbm.at[pl.ds((cid * FH + sid) * VF, VF)],
            table_sh.at[pl.ds(sid * VF, VF)],
            tsem,
        )

    pltpu.sync_copy(x_hbm.at[wid], idx_v)

    lanes = lax.iota(jnp.int32, 16)
    base = wid * RW
    hbase = cid * FH * VF

    def transform(g):
        for k in range(C // 16):
            q = base + g * C + k * 16 + lanes
            l = lax.shift_right_logical(q, 14) - cid * FH
            off = l * VF + jnp.where(l < FS, 0, hbase)
            sl = pl.ds(g * C + k * 16, 16)
            idx_v[sl] = idx_v[sl] + off

    @pl.when(sid < FS)
    def _():
        pltpu.make_async_copy(
            w_hbm.at[pl.ds(0, VF)], table_sh.at[pl.ds(0, VF)], tsem
        ).wait()

    plsc.subcore_barrier()

    def start_gather(g, b):
        l0 = lax.shift_right_logical(base + g * C, 14) - cid * FH

        @pl.when(l0 < FS)
        def _():
            pltpu.async_copy(
                table_sh.at[idx_v.at[pl.ds(g * C, C)]], rows_v.at[b], gsem[b]
            )

        @pl.when(l0 >= FS)
        def _():
            pltpu.async_copy(
                w_hbm.at[idx_v.at[pl.ds(g * C, C)]], rows_v.at[b], gsem[b]
            )

    def wait_gather(b):
        pltpu.make_async_copy(
            w_hbm.at[idx_v.at[pl.ds(0, C)]], rows_v.at[b], gsem[b]
        ).wait()

    def start_out(g, b):
        pltpu.async_copy(
            rows_v.at[b], out_hbm.at[pl.ds(base + g * C, C)], osem[b]
        )

    def wait_out(b):
        pltpu.make_async_copy(
            rows_v.at[b], out_hbm.at[pl.ds(0, C)], osem[b]
        ).wait()

    for b in range(NBUF):
        transform(b)
        start_gather(b, b)

    @pl.loop(0, NCH, step=NBUF)
    def _chunks(g0):
        for b in range(NBUF):
            g = g0 + b
            wait_gather(b)
            start_out(g, b)

            @pl.when(g0 + NBUF < NCH)
            def _():
                transform(g + NBUF)
                wait_out(b)
                start_gather(g + NBUF, b)

    for b in range(NBUF):
        wait_out(b)


def kernel(x, weight):
    xq = x.T.reshape(NW, RW)
    out = _embed(xq, weight)
    return out.reshape(F, B, E).transpose(1, 0, 2)


# speedup vs baseline: 1.0851x; 1.0007x over previous
import functools

import jax
import jax.numpy as jnp
from jax import lax
from jax.experimental import pallas as pl
from jax.experimental.pallas import tpu as pltpu
from jax.experimental.pallas import tpu_sc as plsc

B = 16384
F = 26
E = 128
VF = 1000
R = B * F
NW = 32
RW = R // NW
FH = F // 2
FS = 6
C = 64
NCH = RW // C
NBUF = 8

_mesh = plsc.VectorSubcoreMesh(core_axis_name="c", subcore_axis_name="s")


@functools.partial(
    pl.kernel,
    out_type=jax.ShapeDtypeStruct((R, E), jnp.float32),
    mesh=_mesh,
    scratch_types=[
        pltpu.VMEM((RW,), jnp.int32),
        pltpu.VMEM((NBUF, C, E), jnp.float32),
        pltpu.VMEM_SHARED((FS * VF, E), jnp.float32),
    ]
    + [pltpu.SemaphoreType.DMA] * (2 * NBUF + 1),
)
def _embed(x_hbm, w_hbm, out_hbm, idx_v, rows_v, table_sh, *sems):
    gsem = sems[:NBUF]
    osem = sems[NBUF:2 * NBUF]
    tsem = sems[2 * NBUF]
    cid = lax.axis_index("c")
    sid = lax.axis_index("s")
    wid = cid * 16 + sid

    @pl.when(sid < FS)
    def _():
        pltpu.async_copy(
            w_hbm.at[pl.ds((cid * FH + sid) * VF, VF)],
            table_sh.at[pl.ds(sid * VF, VF)],
            tsem,
        )

    pltpu.sync_copy(x_hbm.at[wid], idx_v)

    lanes = lax.iota(jnp.int32, 16)
    base = wid * RW
    hbase = cid * FH * VF

    def transform(g):
        for k in range(C // 16):
            q = base + g * C + k * 16 + lanes
            l = lax.shift_right_logical(q, 14) - cid * FH
            off = l * VF + jnp.where(l < FS, 0, hbase)
            sl = pl.ds(g * C + k * 16, 16)
            idx_v[sl] = idx_v[sl] + off

    def start_gather_spmem(g, b):
        l0 = lax.shift_right_logical(base + g * C, 14) - cid * FH

        @pl.when(l0 < FS)
        def _():
            pltpu.async_copy(
                table_sh.at[idx_v.at[pl.ds(g * C, C)]], rows_v.at[b], gsem[b]
            )

    def start_gather_hbm(g, b):
        l0 = lax.shift_right_logical(base + g * C, 14) - cid * FH

        @pl.when(l0 >= FS)
        def _():
            pltpu.async_copy(
                w_hbm.at[idx_v.at[pl.ds(g * C, C)]], rows_v.at[b], gsem[b]
            )

    def start_gather(g, b):
        start_gather_spmem(g, b)
        start_gather_hbm(g, b)

    def wait_gather(b):
        pltpu.make_async_copy(
            w_hbm.at[idx_v.at[pl.ds(0, C)]], rows_v.at[b], gsem[b]
        ).wait()

    def start_out(g, b):
        pltpu.async_copy(
            rows_v.at[b], out_hbm.at[pl.ds(base + g * C, C)], osem[b]
        )

    def wait_out(b):
        pltpu.make_async_copy(
            rows_v.at[b], out_hbm.at[pl.ds(0, C)], osem[b]
        ).wait()

    for b in range(NBUF):
        transform(b)
        start_gather_hbm(b, b)

    @pl.when(sid < FS)
    def _():
        pltpu.make_async_copy(
            w_hbm.at[pl.ds(0, VF)], table_sh.at[pl.ds(0, VF)], tsem
        ).wait()

    plsc.subcore_barrier()

    for b in range(NBUF):
        start_gather_spmem(b, b)

    @pl.loop(0, NCH, step=NBUF)
    def _chunks(g0):
        for b in range(NBUF):
            g = g0 + b
            wait_gather(b)
            start_out(g, b)

            @pl.when(g0 + NBUF < NCH)
            def _():
                transform(g + NBUF)
                wait_out(b)
                start_gather(g + NBUF, b)

    for b in range(NBUF):
        wait_out(b)


def kernel(x, weight):
    xq = x.T.reshape(NW, RW)
    out = _embed(xq, weight)
    return out.reshape(F, B, E).transpose(1, 0, 2)
